# Initial kernel scaffold; baseline (speedup 1.0000x reference)
import jax
import jax.numpy as jnp
from jax.experimental import pallas as pl


def kernel(x):
    raise NotImplementedError("write your pallas kernel here")


# speedup vs baseline: 18.0666x; 18.0666x over previous
import jax
import jax.numpy as jnp
from jax.experimental import pallas as pl

H, W_ = 180, 360
HW = H * W_


def _copy_body(x_ref, o_ref):
    o_ref[...] = x_ref[...]


def kernel(x):
    N, C, L = x.shape
    rows = N * C
    x2 = x.reshape(rows, L)
    BLK = 16
    out = pl.pallas_call(
        _copy_body,
        grid=(rows // BLK,),
        in_specs=[pl.BlockSpec((BLK, L), lambda i: (i, 0))],
        out_specs=pl.BlockSpec((BLK, L), lambda i: (i, 0)),
        out_shape=jax.ShapeDtypeStruct((rows, L), x.dtype),
    )(x2)
    return out.reshape(N, C, H, W_)


# speedup vs baseline: 18.1474x; 1.0045x over previous
import jax
import jax.numpy as jnp
from jax.experimental import pallas as pl

H, W_ = 180, 360
HW = H * W_


def _copy_body(x_ref, o_ref):
    o_ref[...] = x_ref[...]


def kernel(x):
    N, C, L = x.shape
    rows = N * C
    x2 = x.reshape(rows, L)
    BLK = 48
    out = pl.pallas_call(
        _copy_body,
        grid=(rows // BLK,),
        in_specs=[pl.BlockSpec((BLK, L), lambda i: (i, 0))],
        out_specs=pl.BlockSpec((BLK, L), lambda i: (i, 0)),
        out_shape=jax.ShapeDtypeStruct((rows, L), x.dtype),
    )(x2)
    return out.reshape(N, C, H, W_)
